# baseline (speedup 1.0000x reference)
import functools

import jax
import jax.numpy as jnp
from jax import lax
from jax.experimental import pallas as pl
from jax.experimental.pallas import tpu as pltpu
from jax.experimental.pallas import tpu_sc as plsc

N = 10000
E = 320000
B = 256
H = 64
ND = 128
SD = 51

NC = 2
NS = 16
NW = NC * NS
EPW = E // NW
CHUNK = 80
NCHUNK = EPW // CHUNK
R0 = 624
R1 = N - R0 * (NS - 1)
DEGW = 8

RB = 1000
NBLK = N // RB


def _sc_mesh():
    return plsc.VectorSubcoreMesh(
        core_axis_name="c", subcore_axis_name="s", num_cores=NC, num_subcores=NS
    )


def _stripe_copy(s, mk_src, mk_dst):
    base = pl.multiple_of(s * R0, 8)

    @pl.when(s < NS - 1)
    def _():
        pltpu.sync_copy(mk_src(base, R0), mk_dst(base, R0))

    @pl.when(s == NS - 1)
    def _():
        pltpu.sync_copy(mk_src(base, R1), mk_dst(base, R1))


def _deg_body(dst_hbm, zeros_hbm, ones_hbm, out_hbm, dst_v, ones_v, acc_sh):
    c = lax.axis_index("c")
    s = lax.axis_index("s")
    w = c * NS + s
    pltpu.sync_copy(dst_hbm.at[w], dst_v)
    pltpu.sync_copy(ones_hbm, ones_v)
    _stripe_copy(s, lambda b, r: zeros_hbm.at[pl.ds(b, r)],
                 lambda b, r: acc_sh.at[pl.ds(b, r)])
    plsc.subcore_barrier()

    def body(j, carry):
        pltpu.sync_copy(ones_v, acc_sh.at[dst_v.at[j]], add=True)
        return carry

    lax.fori_loop(0, NCHUNK, body, 0)
    plsc.subcore_barrier()
    _stripe_copy(s, lambda b, r: acc_sh.at[pl.ds(b, r)],
                 lambda b, r: out_hbm.at[c, pl.ds(b, r)])


def _agg_body(hs_hbm, src_hbm, dst_hbm, zeros_hbm, out_hbm,
              src_v, dst_v, bufa, bufb, acc_sh, sga, sgb, ssa, ssb):
    c = lax.axis_index("c")
    s = lax.axis_index("s")
    w = c * NS + s
    pltpu.sync_copy(src_hbm.at[w], src_v)
    pltpu.sync_copy(dst_hbm.at[w], dst_v)
    _stripe_copy(s, lambda b, r: zeros_hbm.at[pl.ds(b, r)],
                 lambda b, r: acc_sh.at[pl.ds(b, r)])
    plsc.subcore_barrier()

    def gather(j, buf, sem):
        pltpu.async_copy(hs_hbm.at[src_v.at[j]], buf, sem)

    def wait_gather(buf, sem):
        pltpu.make_async_copy(hs_hbm.at[src_v.at[0]], buf, sem).wait()

    def scatter(j, buf, sem):
        pltpu.async_copy(buf, acc_sh.at[dst_v.at[j]], sem, add=True)

    def wait_scatter(buf, sem):
        pltpu.make_async_copy(buf, acc_sh.at[dst_v.at[0]], sem).wait()

    gather(0, bufa, sga)
    gather(1, bufb, sgb)

    def body(i, carry):
        j = 2 * i
        wait_gather(bufa, sga)
        scatter(j, bufa, ssa)
        wait_gather(bufb, sgb)
        scatter(j + 1, bufb, ssb)
        wait_scatter(bufa, ssa)

        @pl.when(j + 2 < NCHUNK)
        def _():
            gather(j + 2, bufa, sga)

        wait_scatter(bufb, ssb)

        @pl.when(j + 3 < NCHUNK)
        def _():
            gather(j + 3, bufb, sgb)

        return carry

    lax.fori_loop(0, NCHUNK // 2, body, 0)
    wait_gather(bufa, sga)
    scatter(NCHUNK - 1, bufa, ssa)
    wait_scatter(bufa, ssa)
    plsc.subcore_barrier()
    _stripe_copy(s, lambda b, r: acc_sh.at[pl.ds(b, r)],
                 lambda b, r: out_hbm.at[c, pl.ds(b, r)])


@functools.lru_cache(maxsize=None)
def _sc_kernels():
    params = pltpu.CompilerParams(use_tc_tiling_on_sc=False)
    deg = pl.kernel(
        _deg_body,
        out_type=jax.ShapeDtypeStruct((NC, N, DEGW), jnp.float32),
        mesh=_sc_mesh(),
        compiler_params=params,
        scratch_types=[
            pltpu.VMEM((NCHUNK, CHUNK), jnp.int32),
            pltpu.VMEM((CHUNK, DEGW), jnp.float32),
            pltpu.VMEM_SHARED((N, DEGW), jnp.float32),
        ],
    )
    agg = pl.kernel(
        _agg_body,
        out_type=jax.ShapeDtypeStruct((NC, N, H), jnp.float32),
        mesh=_sc_mesh(),
        compiler_params=params,
        scratch_types=[
            pltpu.VMEM((NCHUNK, CHUNK), jnp.int32),
            pltpu.VMEM((NCHUNK, CHUNK), jnp.int32),
            pltpu.VMEM((CHUNK, H), jnp.float32),
            pltpu.VMEM((CHUNK, H), jnp.float32),
            pltpu.VMEM_SHARED((N, H), jnp.float32),
            pltpu.SemaphoreType.DMA,
            pltpu.SemaphoreType.DMA,
            pltpu.SemaphoreType.DMA,
            pltpu.SemaphoreType.DMA,
        ],
    )
    return deg, agg


def _dinv_of(degp_ref):
    deg = degp_ref[0, :, 0] + degp_ref[1, :, 0] + 1.0
    return lax.rsqrt(deg)


def _prep_body(degp_ref, x_ref, w1_ref, hs_ref):
    dinv = _dinv_of(degp_ref)
    p = jnp.dot(x_ref[...], w1_ref[...], preferred_element_type=jnp.float32)
    hs_ref[...] = p * dinv[:, None]


_prep = pl.pallas_call(
    _prep_body,
    grid=(NBLK,),
    in_specs=[
        pl.BlockSpec((NC, RB, DEGW), lambda i: (0, i, 0)),
        pl.BlockSpec((RB, ND), lambda i: (i, 0)),
        pl.BlockSpec((ND, H), lambda i: (0, 0)),
    ],
    out_specs=pl.BlockSpec((RB, H), lambda i: (i, 0)),
    out_shape=jax.ShapeDtypeStruct((N, H), jnp.float32),
)


def _mid_body(degp_ref, part_ref, hs_ref, b1_ref, w2_ref, hs2_ref):
    dinv = _dinv_of(degp_ref)
    agg = part_ref[0] + part_ref[1] + hs_ref[...]
    h1 = jnp.maximum(agg * dinv[:, None] + b1_ref[...], 0.0)
    hs2_ref[...] = (
        jnp.dot(h1, w2_ref[...], preferred_element_type=jnp.float32) * dinv[:, None]
    )


_mid = pl.pallas_call(
    _mid_body,
    grid=(NBLK,),
    in_specs=[
        pl.BlockSpec((NC, RB, DEGW), lambda i: (0, i, 0)),
        pl.BlockSpec((NC, RB, H), lambda i: (0, i, 0)),
        pl.BlockSpec((RB, H), lambda i: (i, 0)),
        pl.BlockSpec((1, H), lambda i: (0, 0)),
        pl.BlockSpec((H, H), lambda i: (0, 0)),
    ],
    out_specs=pl.BlockSpec((RB, H), lambda i: (i, 0)),
    out_shape=jax.ShapeDtypeStruct((N, H), jnp.float32),
)


def _final_body(degp_ref, part_ref, hs2_ref, b2_ref, batch_ref, sigma_ref,
                ws1_ref, bs1_ref, ws2_ref, bs2_ref, wf_ref, bf_ref,
                wp_ref, bp_ref, out_ref, acc_ref):
    i = pl.program_id(0)

    @pl.when(i == 0)
    def _():
        acc_ref[...] = jnp.zeros_like(acc_ref)

    dinv = _dinv_of(degp_ref)
    agg = part_ref[0] + part_ref[1] + hs2_ref[...]
    h2 = jnp.maximum(agg * dinv[:, None] + b2_ref[...], 0.0)
    h2e = jnp.concatenate([h2, jnp.ones((RB, 1), jnp.float32)], axis=1)
    bb = batch_ref[0, 0, :]
    onehot = (bb[:, None] == lax.broadcasted_iota(jnp.int32, (1, B), 1)
              ).astype(jnp.float32)
    acc_ref[...] += lax.dot_general(
        onehot, h2e, dimension_numbers=(((0,), (0,)), ((), ())),
        preferred_element_type=jnp.float32)

    @pl.when(i == NBLK - 1)
    def _():
        cnt = jnp.maximum(acc_ref[:, H:H + 1], 1.0)
        mol = acc_ref[:, :H] / cnt
        sf = jnp.maximum(
            jnp.dot(sigma_ref[...], ws1_ref[...],
                    preferred_element_type=jnp.float32) + bs1_ref[...], 0.0)
        sf = jnp.maximum(
            jnp.dot(sf, ws2_ref[...],
                    preferred_element_type=jnp.float32) + bs2_ref[...], 0.0)
        comb = jnp.maximum(
            jnp.dot(sf, wf_ref[0:H, :], preferred_element_type=jnp.float32)
            + jnp.dot(mol, wf_ref[H:2 * H, :], preferred_element_type=jnp.float32)
            + bf_ref[...], 0.0)
        out_ref[...] = (
            jnp.dot(comb, wp_ref[...], preferred_element_type=jnp.float32)
            + bp_ref[...])


_final = pl.pallas_call(
    _final_body,
    grid=(NBLK,),
    in_specs=[
        pl.BlockSpec((NC, RB, DEGW), lambda i: (0, i, 0)),
        pl.BlockSpec((NC, RB, H), lambda i: (0, i, 0)),
        pl.BlockSpec((RB, H), lambda i: (i, 0)),
        pl.BlockSpec((1, H), lambda i: (0, 0)),
        pl.BlockSpec((1, 1, RB), lambda i: (i, 0, 0)),
        pl.BlockSpec((B, SD), lambda i: (0, 0)),
        pl.BlockSpec((SD, 2 * H), lambda i: (0, 0)),
        pl.BlockSpec((1, 2 * H), lambda i: (0, 0)),
        pl.BlockSpec((2 * H, H), lambda i: (0, 0)),
        pl.BlockSpec((1, H), lambda i: (0, 0)),
        pl.BlockSpec((2 * H, H), lambda i: (0, 0)),
        pl.BlockSpec((1, H), lambda i: (0, 0)),
        pl.BlockSpec((H, 1), lambda i: (0, 0)),
        pl.BlockSpec((1, 1), lambda i: (0, 0)),
    ],
    out_specs=pl.BlockSpec((B, 1), lambda i: (0, 0)),
    out_shape=jax.ShapeDtypeStruct((B, 1), jnp.float32),
    scratch_shapes=[pltpu.VMEM((B, H + 1), jnp.float32)],
)


def kernel(sigma_data, x, edge_index, batch, W_g1, b_g1, W_g2, b_g2,
           W_s1, b_s1, W_s2, b_s2, W_f, b_f, W_p, b_p):
    src3 = edge_index[0].reshape(NW, NCHUNK, CHUNK)
    dst3 = edge_index[1].reshape(NW, NCHUNK, CHUNK)
    zeros_h = jnp.zeros((N, H), jnp.float32)
    zeros_d = jnp.zeros((N, DEGW), jnp.float32)
    ones_d = jnp.ones((CHUNK, DEGW), jnp.float32)
    batch3 = batch.reshape(NBLK, 1, RB)

    _deg_kernel, _agg_kernel = _sc_kernels()
    degp = _deg_kernel(dst3, zeros_d, ones_d)
    hs1 = _prep(degp, x, W_g1)
    part1 = _agg_kernel(hs1, src3, dst3, zeros_h)
    hs2 = _mid(degp, part1, hs1, b_g1.reshape(1, H), W_g2)
    part2 = _agg_kernel(hs2, src3, dst3, zeros_h)
    pred = _final(degp, part2, hs2, b_g2.reshape(1, H), batch3, sigma_data,
                  W_s1, b_s1.reshape(1, 2 * H), W_s2, b_s2.reshape(1, H),
                  W_f, b_f.reshape(1, H), W_p, b_p.reshape(1, 1))
    return pred.reshape(B)


# speedup vs baseline: 1.0125x; 1.0125x over previous
import functools

import jax
import jax.numpy as jnp
from jax import lax
from jax.experimental import pallas as pl
from jax.experimental.pallas import tpu as pltpu
from jax.experimental.pallas import tpu_sc as plsc

N = 10000
E = 320000
B = 256
H = 64
ND = 128
SD = 51

NC = 2
NS = 16
NW = NC * NS
EPW = E // NW
CHUNK = 80
NCHUNK = EPW // CHUNK
R0 = 624
R1 = N - R0 * (NS - 1)
DEGW = 8

RB = 1000
NBLK = N // RB


def _sc_mesh():
    return plsc.VectorSubcoreMesh(
        core_axis_name="c", subcore_axis_name="s", num_cores=NC, num_subcores=NS
    )


def _stripe_copy(s, mk_src, mk_dst):
    base = pl.multiple_of(s * R0, 8)

    @pl.when(s < NS - 1)
    def _():
        pltpu.sync_copy(mk_src(base, R0), mk_dst(base, R0))

    @pl.when(s == NS - 1)
    def _():
        pltpu.sync_copy(mk_src(base, R1), mk_dst(base, R1))


def _deg_body(dst_hbm, zeros_hbm, ones_hbm, out_hbm, dst_v, ones_v, acc_sh):
    c = lax.axis_index("c")
    s = lax.axis_index("s")
    w = c * NS + s
    pltpu.sync_copy(dst_hbm.at[w], dst_v)
    pltpu.sync_copy(ones_hbm, ones_v)
    _stripe_copy(s, lambda b, r: zeros_hbm.at[pl.ds(b, r)],
                 lambda b, r: acc_sh.at[pl.ds(b, r)])
    plsc.subcore_barrier()

    def body(j, carry):
        pltpu.sync_copy(ones_v, acc_sh.at[dst_v.at[j]], add=True)
        return carry

    lax.fori_loop(0, NCHUNK, body, 0)
    plsc.subcore_barrier()
    _stripe_copy(s, lambda b, r: acc_sh.at[pl.ds(b, r)],
                 lambda b, r: out_hbm.at[c, pl.ds(b, r)])


NBUF = 4


def _agg_body(hs_hbm, src_hbm, dst_hbm, zeros_hbm, out_hbm,
              src_v, dst_v, *rest):
    bufs = rest[:NBUF]
    hs_sh, acc_sh = rest[NBUF], rest[NBUF + 1]
    sg = rest[NBUF + 2:2 * NBUF + 2]
    ss = rest[2 * NBUF + 2:]
    c = lax.axis_index("c")
    s = lax.axis_index("s")
    w = c * NS + s
    pltpu.sync_copy(src_hbm.at[w], src_v)
    pltpu.sync_copy(dst_hbm.at[w], dst_v)
    _stripe_copy(s, lambda b, r: hs_hbm.at[pl.ds(b, r)],
                 lambda b, r: hs_sh.at[pl.ds(b, r)])
    _stripe_copy(s, lambda b, r: zeros_hbm.at[pl.ds(b, r)],
                 lambda b, r: acc_sh.at[pl.ds(b, r)])
    plsc.subcore_barrier()

    def gather(j, b):
        pltpu.async_copy(hs_sh.at[src_v.at[j]], bufs[b], sg[b])

    def wait_gather(b):
        pltpu.make_async_copy(hs_sh.at[src_v.at[0]], bufs[b], sg[b]).wait()

    def scatter(j, b):
        pltpu.async_copy(bufs[b], acc_sh.at[dst_v.at[j]], ss[b], add=True)

    def wait_scatter(b):
        pltpu.make_async_copy(bufs[b], acc_sh.at[dst_v.at[0]], ss[b]).wait()

    for b in range(NBUF):
        gather(b, b)

    def body(i, carry):
        j = NBUF * i
        for b in range(NBUF):
            wait_gather(b)
            scatter(j + b, b)
        for b in range(NBUF):
            wait_scatter(b)

            @pl.when(j + NBUF + b < NCHUNK)
            def _(b=b):
                gather(j + NBUF + b, b)

        return carry

    lax.fori_loop(0, NCHUNK // NBUF, body, 0)
    for b in range(NCHUNK % NBUF):
        wait_gather(b)
        scatter(NCHUNK - (NCHUNK % NBUF) + b, b)
        wait_scatter(b)
    plsc.subcore_barrier()
    _stripe_copy(s, lambda b, r: acc_sh.at[pl.ds(b, r)],
                 lambda b, r: out_hbm.at[c, pl.ds(b, r)])


@functools.lru_cache(maxsize=None)
def _sc_kernels():
    params = pltpu.CompilerParams(use_tc_tiling_on_sc=False)
    deg = pl.kernel(
        _deg_body,
        out_type=jax.ShapeDtypeStruct((NC, N, DEGW), jnp.float32),
        mesh=_sc_mesh(),
        compiler_params=params,
        scratch_types=[
            pltpu.VMEM((NCHUNK, CHUNK), jnp.int32),
            pltpu.VMEM((CHUNK, DEGW), jnp.float32),
            pltpu.VMEM_SHARED((N, DEGW), jnp.float32),
        ],
    )
    agg = pl.kernel(
        _agg_body,
        out_type=jax.ShapeDtypeStruct((NC, N, H), jnp.float32),
        mesh=_sc_mesh(),
        compiler_params=params,
        scratch_types=(
            [pltpu.VMEM((NCHUNK, CHUNK), jnp.int32)] * 2
            + [pltpu.VMEM((CHUNK, H), jnp.float32)] * NBUF
            + [pltpu.VMEM_SHARED((N, H), jnp.float32)] * 2
            + [pltpu.SemaphoreType.DMA] * (2 * NBUF)
        ),
    )
    return deg, agg


def _dinv_of(degp_ref):
    deg = degp_ref[0, :, 0] + degp_ref[1, :, 0] + 1.0
    return lax.rsqrt(deg)


def _prep_body(degp_ref, x_ref, w1_ref, hs_ref):
    dinv = _dinv_of(degp_ref)
    p = jnp.dot(x_ref[...], w1_ref[...], preferred_element_type=jnp.float32)
    hs_ref[...] = p * dinv[:, None]


_prep = pl.pallas_call(
    _prep_body,
    grid=(NBLK,),
    in_specs=[
        pl.BlockSpec((NC, RB, DEGW), lambda i: (0, i, 0)),
        pl.BlockSpec((RB, ND), lambda i: (i, 0)),
        pl.BlockSpec((ND, H), lambda i: (0, 0)),
    ],
    out_specs=pl.BlockSpec((RB, H), lambda i: (i, 0)),
    out_shape=jax.ShapeDtypeStruct((N, H), jnp.float32),
)


def _mid_body(degp_ref, part_ref, hs_ref, b1_ref, w2_ref, hs2_ref):
    dinv = _dinv_of(degp_ref)
    agg = part_ref[0] + part_ref[1] + hs_ref[...]
    h1 = jnp.maximum(agg * dinv[:, None] + b1_ref[...], 0.0)
    hs2_ref[...] = (
        jnp.dot(h1, w2_ref[...], preferred_element_type=jnp.float32) * dinv[:, None]
    )


_mid = pl.pallas_call(
    _mid_body,
    grid=(NBLK,),
    in_specs=[
        pl.BlockSpec((NC, RB, DEGW), lambda i: (0, i, 0)),
        pl.BlockSpec((NC, RB, H), lambda i: (0, i, 0)),
        pl.BlockSpec((RB, H), lambda i: (i, 0)),
        pl.BlockSpec((1, H), lambda i: (0, 0)),
        pl.BlockSpec((H, H), lambda i: (0, 0)),
    ],
    out_specs=pl.BlockSpec((RB, H), lambda i: (i, 0)),
    out_shape=jax.ShapeDtypeStruct((N, H), jnp.float32),
)


def _final_body(degp_ref, part_ref, hs2_ref, b2_ref, batch_ref, sigma_ref,
                ws1_ref, bs1_ref, ws2_ref, bs2_ref, wf_ref, bf_ref,
                wp_ref, bp_ref, out_ref, acc_ref):
    i = pl.program_id(0)

    @pl.when(i == 0)
    def _():
        acc_ref[...] = jnp.zeros_like(acc_ref)

    dinv = _dinv_of(degp_ref)
    agg = part_ref[0] + part_ref[1] + hs2_ref[...]
    h2 = jnp.maximum(agg * dinv[:, None] + b2_ref[...], 0.0)
    h2e = jnp.concatenate([h2, jnp.ones((RB, 1), jnp.float32)], axis=1)
    bb = batch_ref[0, 0, :]
    onehot = (bb[:, None] == lax.broadcasted_iota(jnp.int32, (1, B), 1)
              ).astype(jnp.float32)
    acc_ref[...] += lax.dot_general(
        onehot, h2e, dimension_numbers=(((0,), (0,)), ((), ())),
        preferred_element_type=jnp.float32)

    @pl.when(i == NBLK - 1)
    def _():
        cnt = jnp.maximum(acc_ref[:, H:H + 1], 1.0)
        mol = acc_ref[:, :H] / cnt
        sf = jnp.maximum(
            jnp.dot(sigma_ref[...], ws1_ref[...],
                    preferred_element_type=jnp.float32) + bs1_ref[...], 0.0)
        sf = jnp.maximum(
            jnp.dot(sf, ws2_ref[...],
                    preferred_element_type=jnp.float32) + bs2_ref[...], 0.0)
        comb = jnp.maximum(
            jnp.dot(sf, wf_ref[0:H, :], preferred_element_type=jnp.float32)
            + jnp.dot(mol, wf_ref[H:2 * H, :], preferred_element_type=jnp.float32)
            + bf_ref[...], 0.0)
        out_ref[...] = (
            jnp.dot(comb, wp_ref[...], preferred_element_type=jnp.float32)
            + bp_ref[...])


_final = pl.pallas_call(
    _final_body,
    grid=(NBLK,),
    in_specs=[
        pl.BlockSpec((NC, RB, DEGW), lambda i: (0, i, 0)),
        pl.BlockSpec((NC, RB, H), lambda i: (0, i, 0)),
        pl.BlockSpec((RB, H), lambda i: (i, 0)),
        pl.BlockSpec((1, H), lambda i: (0, 0)),
        pl.BlockSpec((1, 1, RB), lambda i: (i, 0, 0)),
        pl.BlockSpec((B, SD), lambda i: (0, 0)),
        pl.BlockSpec((SD, 2 * H), lambda i: (0, 0)),
        pl.BlockSpec((1, 2 * H), lambda i: (0, 0)),
        pl.BlockSpec((2 * H, H), lambda i: (0, 0)),
        pl.BlockSpec((1, H), lambda i: (0, 0)),
        pl.BlockSpec((2 * H, H), lambda i: (0, 0)),
        pl.BlockSpec((1, H), lambda i: (0, 0)),
        pl.BlockSpec((H, 1), lambda i: (0, 0)),
        pl.BlockSpec((1, 1), lambda i: (0, 0)),
    ],
    out_specs=pl.BlockSpec((B, 1), lambda i: (0, 0)),
    out_shape=jax.ShapeDtypeStruct((B, 1), jnp.float32),
    scratch_shapes=[pltpu.VMEM((B, H + 1), jnp.float32)],
)


def kernel(sigma_data, x, edge_index, batch, W_g1, b_g1, W_g2, b_g2,
           W_s1, b_s1, W_s2, b_s2, W_f, b_f, W_p, b_p):
    src3 = edge_index[0].reshape(NW, NCHUNK, CHUNK)
    dst3 = edge_index[1].reshape(NW, NCHUNK, CHUNK)
    zeros_h = jnp.zeros((N, H), jnp.float32)
    zeros_d = jnp.zeros((N, DEGW), jnp.float32)
    ones_d = jnp.ones((CHUNK, DEGW), jnp.float32)
    batch3 = batch.reshape(NBLK, 1, RB)

    _deg_kernel, _agg_kernel = _sc_kernels()
    degp = _deg_kernel(dst3, zeros_d, ones_d)
    hs1 = _prep(degp, x, W_g1)
    part1 = _agg_kernel(hs1, src3, dst3, zeros_h)
    hs2 = _mid(degp, part1, hs1, b_g1.reshape(1, H), W_g2)
    part2 = _agg_kernel(hs2, src3, dst3, zeros_h)
    pred = _final(degp, part2, hs2, b_g2.reshape(1, H), batch3, sigma_data,
                  W_s1, b_s1.reshape(1, 2 * H), W_s2, b_s2.reshape(1, H),
                  W_f, b_f.reshape(1, H), W_p, b_p.reshape(1, 1))
    return pred.reshape(B)


# speedup vs baseline: 1.0689x; 1.0557x over previous
import functools

import jax
import jax.numpy as jnp
from jax import lax
from jax.experimental import pallas as pl
from jax.experimental.pallas import tpu as pltpu
from jax.experimental.pallas import tpu_sc as plsc

N = 10000
E = 320000
B = 256
H = 64
ND = 128
SD = 51

NC = 2
NS = 16
NW = NC * NS
EPW = E // NW
CHUNK = 80
NCHUNK = EPW // CHUNK
R0 = 624
R1 = N - R0 * (NS - 1)
DEGW = 8

RB = 1000
NBLK = N // RB


def _sc_mesh():
    return plsc.VectorSubcoreMesh(
        core_axis_name="c", subcore_axis_name="s", num_cores=NC, num_subcores=NS
    )


def _stripe_copy(s, mk_src, mk_dst):
    base = pl.multiple_of(s * R0, 8)

    @pl.when(s < NS - 1)
    def _():
        pltpu.sync_copy(mk_src(base, R0), mk_dst(base, R0))

    @pl.when(s == NS - 1)
    def _():
        pltpu.sync_copy(mk_src(base, R1), mk_dst(base, R1))


def _deg_body(dst_hbm, zeros_hbm, ones_hbm, out_hbm, dst_v, ones_v, acc_sh):
    c = lax.axis_index("c")
    s = lax.axis_index("s")
    w = c * NS + s
    pltpu.sync_copy(dst_hbm.at[w], dst_v)
    pltpu.sync_copy(ones_hbm, ones_v)
    _stripe_copy(s, lambda b, r: zeros_hbm.at[pl.ds(b, r)],
                 lambda b, r: acc_sh.at[pl.ds(b, r)])
    plsc.subcore_barrier()

    def body(j, carry):
        pltpu.sync_copy(ones_v, acc_sh.at[dst_v.at[j]], add=True)
        return carry

    lax.fori_loop(0, NCHUNK, body, 0)
    plsc.subcore_barrier()
    _stripe_copy(s, lambda b, r: acc_sh.at[pl.ds(b, r)],
                 lambda b, r: out_hbm.at[c, pl.ds(b, r)])


def _agg_body(hs_hbm, src_hbm, dst_hbm, zeros_hbm, out_hbm,
              src_v, dst_v, bufa, bufb, hs_sh, acc_sh, sga, sgb, ssa, ssb):
    c = lax.axis_index("c")
    s = lax.axis_index("s")
    w = c * NS + s
    pltpu.sync_copy(src_hbm.at[w], src_v)
    pltpu.sync_copy(dst_hbm.at[w], dst_v)
    _stripe_copy(s, lambda b, r: hs_hbm.at[pl.ds(b, r)],
                 lambda b, r: hs_sh.at[pl.ds(b, r)])
    _stripe_copy(s, lambda b, r: zeros_hbm.at[pl.ds(b, r)],
                 lambda b, r: acc_sh.at[pl.ds(b, r)])
    plsc.subcore_barrier()

    def gather(j, buf, sem):
        pltpu.async_copy(hs_sh.at[src_v.at[j]], buf, sem)

    def wait_gather(buf, sem):
        pltpu.make_async_copy(hs_sh.at[src_v.at[0]], buf, sem).wait()

    def scatter(j, buf, sem):
        pltpu.async_copy(buf, acc_sh.at[dst_v.at[j]], sem, add=True)

    def wait_scatter(buf, sem):
        pltpu.make_async_copy(buf, acc_sh.at[dst_v.at[0]], sem).wait()

    gather(0, bufa, sga)
    gather(1, bufb, sgb)

    def body(i, carry):
        j = 2 * i
        wait_gather(bufa, sga)
        scatter(j, bufa, ssa)
        wait_gather(bufb, sgb)
        scatter(j + 1, bufb, ssb)
        wait_scatter(bufa, ssa)

        @pl.when(j + 2 < NCHUNK)
        def _():
            gather(j + 2, bufa, sga)

        wait_scatter(bufb, ssb)

        @pl.when(j + 3 < NCHUNK)
        def _():
            gather(j + 3, bufb, sgb)

        return carry

    lax.fori_loop(0, NCHUNK // 2, body, 0)
    wait_gather(bufa, sga)
    scatter(NCHUNK - 1, bufa, ssa)
    wait_scatter(bufa, ssa)
    plsc.subcore_barrier()
    _stripe_copy(s, lambda b, r: acc_sh.at[pl.ds(b, r)],
                 lambda b, r: out_hbm.at[c, pl.ds(b, r)])


@functools.lru_cache(maxsize=None)
def _sc_kernels():
    params = pltpu.CompilerParams(use_tc_tiling_on_sc=False)
    deg = pl.kernel(
        _deg_body,
        out_type=jax.ShapeDtypeStruct((NC, N, DEGW), jnp.float32),
        mesh=_sc_mesh(),
        compiler_params=params,
        scratch_types=[
            pltpu.VMEM((NCHUNK, CHUNK), jnp.int32),
            pltpu.VMEM((CHUNK, DEGW), jnp.float32),
            pltpu.VMEM_SHARED((N, DEGW), jnp.float32),
        ],
    )
    agg = pl.kernel(
        _agg_body,
        out_type=jax.ShapeDtypeStruct((NC, N, H), jnp.float32),
        mesh=_sc_mesh(),
        compiler_params=params,
        scratch_types=[
            pltpu.VMEM((NCHUNK, CHUNK), jnp.int32),
            pltpu.VMEM((NCHUNK, CHUNK), jnp.int32),
            pltpu.VMEM((CHUNK, H), jnp.float32),
            pltpu.VMEM((CHUNK, H), jnp.float32),
            pltpu.VMEM_SHARED((N, H), jnp.float32),
            pltpu.VMEM_SHARED((N, H), jnp.float32),
            pltpu.SemaphoreType.DMA,
            pltpu.SemaphoreType.DMA,
            pltpu.SemaphoreType.DMA,
            pltpu.SemaphoreType.DMA,
        ],
    )
    return deg, agg


def _dinv_of(degp_ref):
    deg = degp_ref[0, :, 0] + degp_ref[1, :, 0] + 1.0
    return lax.rsqrt(deg)


def _prep_body(degp_ref, x_ref, w1_ref, hs_ref):
    dinv = _dinv_of(degp_ref)
    p = jnp.dot(x_ref[...], w1_ref[...], preferred_element_type=jnp.float32)
    hs_ref[...] = p * dinv[:, None]


_prep = pl.pallas_call(
    _prep_body,
    grid=(NBLK,),
    in_specs=[
        pl.BlockSpec((NC, RB, DEGW), lambda i: (0, i, 0)),
        pl.BlockSpec((RB, ND), lambda i: (i, 0)),
        pl.BlockSpec((ND, H), lambda i: (0, 0)),
    ],
    out_specs=pl.BlockSpec((RB, H), lambda i: (i, 0)),
    out_shape=jax.ShapeDtypeStruct((N, H), jnp.float32),
)


def _mid_body(degp_ref, part_ref, hs_ref, b1_ref, w2_ref, hs2_ref):
    dinv = _dinv_of(degp_ref)
    agg = part_ref[0] + part_ref[1] + hs_ref[...]
    h1 = jnp.maximum(agg * dinv[:, None] + b1_ref[...], 0.0)
    hs2_ref[...] = (
        jnp.dot(h1, w2_ref[...], preferred_element_type=jnp.float32) * dinv[:, None]
    )


_mid = pl.pallas_call(
    _mid_body,
    grid=(NBLK,),
    in_specs=[
        pl.BlockSpec((NC, RB, DEGW), lambda i: (0, i, 0)),
        pl.BlockSpec((NC, RB, H), lambda i: (0, i, 0)),
        pl.BlockSpec((RB, H), lambda i: (i, 0)),
        pl.BlockSpec((1, H), lambda i: (0, 0)),
        pl.BlockSpec((H, H), lambda i: (0, 0)),
    ],
    out_specs=pl.BlockSpec((RB, H), lambda i: (i, 0)),
    out_shape=jax.ShapeDtypeStruct((N, H), jnp.float32),
)


def _final_body(degp_ref, part_ref, hs2_ref, b2_ref, batch_ref, sigma_ref,
                ws1_ref, bs1_ref, ws2_ref, bs2_ref, wf_ref, bf_ref,
                wp_ref, bp_ref, out_ref, acc_ref):
    i = pl.program_id(0)

    @pl.when(i == 0)
    def _():
        acc_ref[...] = jnp.zeros_like(acc_ref)

    dinv = _dinv_of(degp_ref)
    agg = part_ref[0] + part_ref[1] + hs2_ref[...]
    h2 = jnp.maximum(agg * dinv[:, None] + b2_ref[...], 0.0)
    h2e = jnp.concatenate([h2, jnp.ones((RB, 1), jnp.float32)], axis=1)
    bb = batch_ref[0, 0, :]
    onehot = (bb[:, None] == lax.broadcasted_iota(jnp.int32, (1, B), 1)
              ).astype(jnp.float32)
    acc_ref[...] += lax.dot_general(
        onehot, h2e, dimension_numbers=(((0,), (0,)), ((), ())),
        preferred_element_type=jnp.float32)

    @pl.when(i == NBLK - 1)
    def _():
        cnt = jnp.maximum(acc_ref[:, H:H + 1], 1.0)
        mol = acc_ref[:, :H] / cnt
        sf = jnp.maximum(
            jnp.dot(sigma_ref[...], ws1_ref[...],
                    preferred_element_type=jnp.float32) + bs1_ref[...], 0.0)
        sf = jnp.maximum(
            jnp.dot(sf, ws2_ref[...],
                    preferred_element_type=jnp.float32) + bs2_ref[...], 0.0)
        comb = jnp.maximum(
            jnp.dot(sf, wf_ref[0:H, :], preferred_element_type=jnp.float32)
            + jnp.dot(mol, wf_ref[H:2 * H, :], preferred_element_type=jnp.float32)
            + bf_ref[...], 0.0)
        out_ref[...] = (
            jnp.dot(comb, wp_ref[...], preferred_element_type=jnp.float32)
            + bp_ref[...])


_final = pl.pallas_call(
    _final_body,
    grid=(NBLK,),
    in_specs=[
        pl.BlockSpec((NC, RB, DEGW), lambda i: (0, i, 0)),
        pl.BlockSpec((NC, RB, H), lambda i: (0, i, 0)),
        pl.BlockSpec((RB, H), lambda i: (i, 0)),
        pl.BlockSpec((1, H), lambda i: (0, 0)),
        pl.BlockSpec((1, 1, RB), lambda i: (i, 0, 0)),
        pl.BlockSpec((B, SD), lambda i: (0, 0)),
        pl.BlockSpec((SD, 2 * H), lambda i: (0, 0)),
        pl.BlockSpec((1, 2 * H), lambda i: (0, 0)),
        pl.BlockSpec((2 * H, H), lambda i: (0, 0)),
        pl.BlockSpec((1, H), lambda i: (0, 0)),
        pl.BlockSpec((2 * H, H), lambda i: (0, 0)),
        pl.BlockSpec((1, H), lambda i: (0, 0)),
        pl.BlockSpec((H, 1), lambda i: (0, 0)),
        pl.BlockSpec((1, 1), lambda i: (0, 0)),
    ],
    out_specs=pl.BlockSpec((B, 1), lambda i: (0, 0)),
    out_shape=jax.ShapeDtypeStruct((B, 1), jnp.float32),
    scratch_shapes=[pltpu.VMEM((B, H + 1), jnp.float32)],
)


def kernel(sigma_data, x, edge_index, batch, W_g1, b_g1, W_g2, b_g2,
           W_s1, b_s1, W_s2, b_s2, W_f, b_f, W_p, b_p):
    src3 = edge_index[0].reshape(NW, NCHUNK, CHUNK)
    dst3 = edge_index[1].reshape(NW, NCHUNK, CHUNK)
    zeros_h = jnp.zeros((N, H), jnp.float32)
    zeros_d = jnp.zeros((N, DEGW), jnp.float32)
    ones_d = jnp.ones((CHUNK, DEGW), jnp.float32)
    batch3 = batch.reshape(NBLK, 1, RB)

    _deg_kernel, _agg_kernel = _sc_kernels()
    degp = _deg_kernel(dst3, zeros_d, ones_d)
    hs1 = _prep(degp, x, W_g1)
    part1 = _agg_kernel(hs1, src3, dst3, zeros_h)
    hs2 = _mid(degp, part1, hs1, b_g1.reshape(1, H), W_g2)
    part2 = _agg_kernel(hs2, src3, dst3, zeros_h)
    pred = _final(degp, part2, hs2, b_g2.reshape(1, H), batch3, sigma_data,
                  W_s1, b_s1.reshape(1, 2 * H), W_s2, b_s2.reshape(1, H),
                  W_f, b_f.reshape(1, H), W_p, b_p.reshape(1, 1))
    return pred.reshape(B)


# speedup vs baseline: 1.0825x; 1.0127x over previous
import functools

import jax
import jax.numpy as jnp
from jax import lax
from jax.experimental import pallas as pl
from jax.experimental.pallas import tpu as pltpu
from jax.experimental.pallas import tpu_sc as plsc

N = 10000
E = 320000
B = 256
H = 64
ND = 128
SD = 51

NC = 2
NS = 16
NW = NC * NS
EPW = E // NW
CHUNK = 128
NCHUNK = -(-EPW // CHUNK)
EPAD = NCHUNK * CHUNK
NPAD = 10008
SINK = N
R0 = 624
R1 = N - R0 * (NS - 1)
RZ = NPAD - R0 * (NS - 1)
DEGW = 8

RB = 1000
NBLK = N // RB


def _sc_mesh():
    return plsc.VectorSubcoreMesh(
        core_axis_name="c", subcore_axis_name="s", num_cores=NC, num_subcores=NS
    )


def _stripe_copy(s, mk_src, mk_dst, last=R1):
    base = pl.multiple_of(s * R0, 8)

    @pl.when(s < NS - 1)
    def _():
        pltpu.sync_copy(mk_src(base, R0), mk_dst(base, R0))

    @pl.when(s == NS - 1)
    def _():
        pltpu.sync_copy(mk_src(base, last), mk_dst(base, last))


def _deg_body(dst_hbm, zeros_hbm, ones_hbm, out_hbm, dst_v, ones_v, acc_sh, sem):
    c = lax.axis_index("c")
    s = lax.axis_index("s")
    w = c * NS + s
    pltpu.sync_copy(dst_hbm.at[w], dst_v)
    pltpu.sync_copy(ones_hbm, ones_v)
    _stripe_copy(s, lambda b, r: zeros_hbm.at[pl.ds(b, r)],
                 lambda b, r: acc_sh.at[pl.ds(b, r)], last=RZ)
    plsc.subcore_barrier()

    def body(j, carry):
        pltpu.async_copy(ones_v, acc_sh.at[dst_v.at[j]], sem, add=True)
        return carry

    lax.fori_loop(0, NCHUNK, body, 0)

    def drain(j, carry):
        pltpu.make_async_copy(ones_v, acc_sh.at[dst_v.at[0]], sem).wait()
        return carry

    lax.fori_loop(0, NCHUNK, drain, 0)
    plsc.subcore_barrier()
    _stripe_copy(s, lambda b, r: acc_sh.at[pl.ds(b, r)],
                 lambda b, r: out_hbm.at[c, pl.ds(b, r)])


def _agg_body(hs_hbm, src_hbm, dst_hbm, zeros_hbm, out_hbm,
              src_v, dst_v, bufa, bufb, hs_sh, acc_sh, sga, sgb, ssa, ssb):
    c = lax.axis_index("c")
    s = lax.axis_index("s")
    w = c * NS + s
    pltpu.sync_copy(src_hbm.at[w], src_v)
    pltpu.sync_copy(dst_hbm.at[w], dst_v)
    _stripe_copy(s, lambda b, r: hs_hbm.at[pl.ds(b, r)],
                 lambda b, r: hs_sh.at[pl.ds(b, r)])
    _stripe_copy(s, lambda b, r: zeros_hbm.at[pl.ds(b, r)],
                 lambda b, r: acc_sh.at[pl.ds(b, r)], last=RZ)
    plsc.subcore_barrier()

    def gather(j, buf, sem):
        pltpu.async_copy(hs_sh.at[src_v.at[j]], buf, sem)

    def wait_gather(buf, sem):
        pltpu.make_async_copy(hs_sh.at[src_v.at[0]], buf, sem).wait()

    def scatter(j, buf, sem):
        pltpu.async_copy(buf, acc_sh.at[dst_v.at[j]], sem, add=True)

    def wait_scatter(buf, sem):
        pltpu.make_async_copy(buf, acc_sh.at[dst_v.at[0]], sem).wait()

    gather(0, bufa, sga)
    gather(1, bufb, sgb)

    def body(i, carry):
        j = 2 * i
        wait_gather(bufa, sga)
        scatter(j, bufa, ssa)
        wait_gather(bufb, sgb)
        scatter(j + 1, bufb, ssb)
        wait_scatter(bufa, ssa)

        @pl.when(j + 2 < NCHUNK)
        def _():
            gather(j + 2, bufa, sga)

        wait_scatter(bufb, ssb)

        @pl.when(j + 3 < NCHUNK)
        def _():
            gather(j + 3, bufb, sgb)

        return carry

    lax.fori_loop(0, NCHUNK // 2, body, 0)
    wait_gather(bufa, sga)
    scatter(NCHUNK - 1, bufa, ssa)
    wait_scatter(bufa, ssa)
    plsc.subcore_barrier()
    _stripe_copy(s, lambda b, r: acc_sh.at[pl.ds(b, r)],
                 lambda b, r: out_hbm.at[c, pl.ds(b, r)])


@functools.lru_cache(maxsize=None)
def _sc_kernels():
    params = pltpu.CompilerParams(use_tc_tiling_on_sc=False)
    deg = pl.kernel(
        _deg_body,
        out_type=jax.ShapeDtypeStruct((NC, N, DEGW), jnp.float32),
        mesh=_sc_mesh(),
        compiler_params=params,
        scratch_types=[
            pltpu.VMEM((NCHUNK, CHUNK), jnp.int32),
            pltpu.VMEM((CHUNK, DEGW), jnp.float32),
            pltpu.VMEM_SHARED((NPAD, DEGW), jnp.float32),
            pltpu.SemaphoreType.DMA,
        ],
    )
    agg = pl.kernel(
        _agg_body,
        out_type=jax.ShapeDtypeStruct((NC, N, H), jnp.float32),
        mesh=_sc_mesh(),
        compiler_params=params,
        scratch_types=[
            pltpu.VMEM((NCHUNK, CHUNK), jnp.int32),
            pltpu.VMEM((NCHUNK, CHUNK), jnp.int32),
            pltpu.VMEM((CHUNK, H), jnp.float32),
            pltpu.VMEM((CHUNK, H), jnp.float32),
            pltpu.VMEM_SHARED((N, H), jnp.float32),
            pltpu.VMEM_SHARED((NPAD, H), jnp.float32),
            pltpu.SemaphoreType.DMA,
            pltpu.SemaphoreType.DMA,
            pltpu.SemaphoreType.DMA,
            pltpu.SemaphoreType.DMA,
        ],
    )
    return deg, agg


def _dinv_of(degp_ref):
    deg = degp_ref[0, :, 0] + degp_ref[1, :, 0] + 1.0
    return lax.rsqrt(deg)


def _prep_body(degp_ref, x_ref, w1_ref, hs_ref):
    dinv = _dinv_of(degp_ref)
    p = jnp.dot(x_ref[...], w1_ref[...], preferred_element_type=jnp.float32)
    hs_ref[...] = p * dinv[:, None]


_prep = pl.pallas_call(
    _prep_body,
    grid=(NBLK,),
    in_specs=[
        pl.BlockSpec((NC, RB, DEGW), lambda i: (0, i, 0)),
        pl.BlockSpec((RB, ND), lambda i: (i, 0)),
        pl.BlockSpec((ND, H), lambda i: (0, 0)),
    ],
    out_specs=pl.BlockSpec((RB, H), lambda i: (i, 0)),
    out_shape=jax.ShapeDtypeStruct((N, H), jnp.float32),
)


def _mid_body(degp_ref, part_ref, hs_ref, b1_ref, w2_ref, hs2_ref):
    dinv = _dinv_of(degp_ref)
    agg = part_ref[0] + part_ref[1] + hs_ref[...]
    h1 = jnp.maximum(agg * dinv[:, None] + b1_ref[...], 0.0)
    hs2_ref[...] = (
        jnp.dot(h1, w2_ref[...], preferred_element_type=jnp.float32) * dinv[:, None]
    )


_mid = pl.pallas_call(
    _mid_body,
    grid=(NBLK,),
    in_specs=[
        pl.BlockSpec((NC, RB, DEGW), lambda i: (0, i, 0)),
        pl.BlockSpec((NC, RB, H), lambda i: (0, i, 0)),
        pl.BlockSpec((RB, H), lambda i: (i, 0)),
        pl.BlockSpec((1, H), lambda i: (0, 0)),
        pl.BlockSpec((H, H), lambda i: (0, 0)),
    ],
    out_specs=pl.BlockSpec((RB, H), lambda i: (i, 0)),
    out_shape=jax.ShapeDtypeStruct((N, H), jnp.float32),
)


def _final_body(degp_ref, part_ref, hs2_ref, b2_ref, batch_ref, sigma_ref,
                ws1_ref, bs1_ref, ws2_ref, bs2_ref, wf_ref, bf_ref,
                wp_ref, bp_ref, out_ref, acc_ref):
    i = pl.program_id(0)

    @pl.when(i == 0)
    def _():
        acc_ref[...] = jnp.zeros_like(acc_ref)

    dinv = _dinv_of(degp_ref)
    agg = part_ref[0] + part_ref[1] + hs2_ref[...]
    h2 = jnp.maximum(agg * dinv[:, None] + b2_ref[...], 0.0)
    h2e = jnp.concatenate([h2, jnp.ones((RB, 1), jnp.float32)], axis=1)
    bb = batch_ref[0, 0, :]
    onehot = (bb[:, None] == lax.broadcasted_iota(jnp.int32, (1, B), 1)
              ).astype(jnp.float32)
    acc_ref[...] += lax.dot_general(
        onehot, h2e, dimension_numbers=(((0,), (0,)), ((), ())),
        preferred_element_type=jnp.float32)

    @pl.when(i == NBLK - 1)
    def _():
        cnt = jnp.maximum(acc_ref[:, H:H + 1], 1.0)
        mol = acc_ref[:, :H] / cnt
        sf = jnp.maximum(
            jnp.dot(sigma_ref[...], ws1_ref[...],
                    preferred_element_type=jnp.float32) + bs1_ref[...], 0.0)
        sf = jnp.maximum(
            jnp.dot(sf, ws2_ref[...],
                    preferred_element_type=jnp.float32) + bs2_ref[...], 0.0)
        comb = jnp.maximum(
            jnp.dot(sf, wf_ref[0:H, :], preferred_element_type=jnp.float32)
            + jnp.dot(mol, wf_ref[H:2 * H, :], preferred_element_type=jnp.float32)
            + bf_ref[...], 0.0)
        out_ref[...] = (
            jnp.dot(comb, wp_ref[...], preferred_element_type=jnp.float32)
            + bp_ref[...])


_final = pl.pallas_call(
    _final_body,
    grid=(NBLK,),
    in_specs=[
        pl.BlockSpec((NC, RB, DEGW), lambda i: (0, i, 0)),
        pl.BlockSpec((NC, RB, H), lambda i: (0, i, 0)),
        pl.BlockSpec((RB, H), lambda i: (i, 0)),
        pl.BlockSpec((1, H), lambda i: (0, 0)),
        pl.BlockSpec((1, 1, RB), lambda i: (i, 0, 0)),
        pl.BlockSpec((B, SD), lambda i: (0, 0)),
        pl.BlockSpec((SD, 2 * H), lambda i: (0, 0)),
        pl.BlockSpec((1, 2 * H), lambda i: (0, 0)),
        pl.BlockSpec((2 * H, H), lambda i: (0, 0)),
        pl.BlockSpec((1, H), lambda i: (0, 0)),
        pl.BlockSpec((2 * H, H), lambda i: (0, 0)),
        pl.BlockSpec((1, H), lambda i: (0, 0)),
        pl.BlockSpec((H, 1), lambda i: (0, 0)),
        pl.BlockSpec((1, 1), lambda i: (0, 0)),
    ],
    out_specs=pl.BlockSpec((B, 1), lambda i: (0, 0)),
    out_shape=jax.ShapeDtypeStruct((B, 1), jnp.float32),
    scratch_shapes=[pltpu.VMEM((B, H + 1), jnp.float32)],
)


def kernel(sigma_data, x, edge_index, batch, W_g1, b_g1, W_g2, b_g2,
           W_s1, b_s1, W_s2, b_s2, W_f, b_f, W_p, b_p):
    src2 = edge_index[0].reshape(NW, EPW)
    dst2 = edge_index[1].reshape(NW, EPW)
    pad = ((0, 0), (0, EPAD - EPW))
    src3 = jnp.pad(src2, pad).reshape(NW, NCHUNK, CHUNK)
    dst3 = jnp.pad(dst2, pad, constant_values=SINK).reshape(NW, NCHUNK, CHUNK)
    zeros_h = jnp.zeros((NPAD, H), jnp.float32)
    zeros_d = jnp.zeros((NPAD, DEGW), jnp.float32)
    ones_d = jnp.ones((CHUNK, DEGW), jnp.float32)
    batch3 = batch.reshape(NBLK, 1, RB)

    _deg_kernel, _agg_kernel = _sc_kernels()
    degp = _deg_kernel(dst3, zeros_d, ones_d)
    hs1 = _prep(degp, x, W_g1)
    part1 = _agg_kernel(hs1, src3, dst3, zeros_h)
    hs2 = _mid(degp, part1, hs1, b_g1.reshape(1, H), W_g2)
    part2 = _agg_kernel(hs2, src3, dst3, zeros_h)
    pred = _final(degp, part2, hs2, b_g2.reshape(1, H), batch3, sigma_data,
                  W_s1, b_s1.reshape(1, 2 * H), W_s2, b_s2.reshape(1, H),
                  W_f, b_f.reshape(1, H), W_p, b_p.reshape(1, 1))
    return pred.reshape(B)


# speedup vs baseline: 1.1093x; 1.0248x over previous
import functools

import jax
import jax.numpy as jnp
from jax import lax
from jax.experimental import pallas as pl
from jax.experimental.pallas import tpu as pltpu
from jax.experimental.pallas import tpu_sc as plsc

N = 10000
E = 320000
B = 256
H = 64
ND = 128
SD = 51

NC = 2
NS = 16
NW = NC * NS
EPW = E // NW
CHUNK = 128
NCHUNK = -(-EPW // CHUNK)
EPAD = NCHUNK * CHUNK
NPAD = 10008
SINK = N
R0 = 624
R1 = N - R0 * (NS - 1)
RZ = NPAD - R0 * (NS - 1)
DEGW = 8

RB = 2000
NBLK = N // RB


def _sc_mesh():
    return plsc.VectorSubcoreMesh(
        core_axis_name="c", subcore_axis_name="s", num_cores=NC, num_subcores=NS
    )


def _stripe_copy(s, mk_src, mk_dst, last=R1):
    base = pl.multiple_of(s * R0, 8)

    @pl.when(s < NS - 1)
    def _():
        pltpu.sync_copy(mk_src(base, R0), mk_dst(base, R0))

    @pl.when(s == NS - 1)
    def _():
        pltpu.sync_copy(mk_src(base, last), mk_dst(base, last))


def _deg_body(dst_hbm, zeros_hbm, ones_hbm, out_hbm, dst_v, ones_v, acc_sh, sem):
    c = lax.axis_index("c")
    s = lax.axis_index("s")
    w = c * NS + s
    pltpu.sync_copy(dst_hbm.at[w], dst_v)
    pltpu.sync_copy(ones_hbm, ones_v)
    _stripe_copy(s, lambda b, r: zeros_hbm.at[pl.ds(b, r)],
                 lambda b, r: acc_sh.at[pl.ds(b, r)], last=RZ)
    plsc.subcore_barrier()

    def body(j, carry):
        pltpu.async_copy(ones_v, acc_sh.at[dst_v.at[j]], sem, add=True)
        return carry

    lax.fori_loop(0, NCHUNK, body, 0)

    def drain(j, carry):
        pltpu.make_async_copy(ones_v, acc_sh.at[dst_v.at[0]], sem).wait()
        return carry

    lax.fori_loop(0, NCHUNK, drain, 0)
    plsc.subcore_barrier()
    _stripe_copy(s, lambda b, r: acc_sh.at[pl.ds(b, r)],
                 lambda b, r: out_hbm.at[c, pl.ds(b, r)])


def _agg_body(hs_hbm, src_hbm, dst_hbm, zeros_hbm, out_hbm,
              src_v, dst_v, bufa, bufb, hs_sh, acc_sh, sga, sgb, ssa, ssb):
    c = lax.axis_index("c")
    s = lax.axis_index("s")
    w = c * NS + s
    pltpu.sync_copy(src_hbm.at[w], src_v)
    pltpu.sync_copy(dst_hbm.at[w], dst_v)
    _stripe_copy(s, lambda b, r: hs_hbm.at[pl.ds(b, r)],
                 lambda b, r: hs_sh.at[pl.ds(b, r)])
    _stripe_copy(s, lambda b, r: zeros_hbm.at[pl.ds(b, r)],
                 lambda b, r: acc_sh.at[pl.ds(b, r)], last=RZ)
    plsc.subcore_barrier()

    def gather(j, buf, sem):
        pltpu.async_copy(hs_sh.at[src_v.at[j]], buf, sem)

    def wait_gather(buf, sem):
        pltpu.make_async_copy(hs_sh.at[src_v.at[0]], buf, sem).wait()

    def scatter(j, buf, sem):
        pltpu.async_copy(buf, acc_sh.at[dst_v.at[j]], sem, add=True)

    def wait_scatter(buf, sem):
        pltpu.make_async_copy(buf, acc_sh.at[dst_v.at[0]], sem).wait()

    gather(0, bufa, sga)
    gather(1, bufb, sgb)

    def body(i, carry):
        j = 2 * i
        wait_gather(bufa, sga)
        scatter(j, bufa, ssa)
        wait_gather(bufb, sgb)
        scatter(j + 1, bufb, ssb)
        wait_scatter(bufa, ssa)

        @pl.when(j + 2 < NCHUNK)
        def _():
            gather(j + 2, bufa, sga)

        wait_scatter(bufb, ssb)

        @pl.when(j + 3 < NCHUNK)
        def _():
            gather(j + 3, bufb, sgb)

        return carry

    lax.fori_loop(0, NCHUNK // 2, body, 0)
    wait_gather(bufa, sga)
    scatter(NCHUNK - 1, bufa, ssa)
    wait_scatter(bufa, ssa)
    plsc.subcore_barrier()
    _stripe_copy(s, lambda b, r: acc_sh.at[pl.ds(b, r)],
                 lambda b, r: out_hbm.at[c, pl.ds(b, r)])


@functools.lru_cache(maxsize=None)
def _sc_kernels():
    params = pltpu.CompilerParams(use_tc_tiling_on_sc=False)
    deg = pl.kernel(
        _deg_body,
        out_type=jax.ShapeDtypeStruct((NC, N, DEGW), jnp.float32),
        mesh=_sc_mesh(),
        compiler_params=params,
        scratch_types=[
            pltpu.VMEM((NCHUNK, CHUNK), jnp.int32),
            pltpu.VMEM((CHUNK, DEGW), jnp.float32),
            pltpu.VMEM_SHARED((NPAD, DEGW), jnp.float32),
            pltpu.SemaphoreType.DMA,
        ],
    )
    agg = pl.kernel(
        _agg_body,
        out_type=jax.ShapeDtypeStruct((NC, N, H), jnp.float32),
        mesh=_sc_mesh(),
        compiler_params=params,
        scratch_types=[
            pltpu.VMEM((NCHUNK, CHUNK), jnp.int32),
            pltpu.VMEM((NCHUNK, CHUNK), jnp.int32),
            pltpu.VMEM((CHUNK, H), jnp.float32),
            pltpu.VMEM((CHUNK, H), jnp.float32),
            pltpu.VMEM_SHARED((N, H), jnp.float32),
            pltpu.VMEM_SHARED((NPAD, H), jnp.float32),
            pltpu.SemaphoreType.DMA,
            pltpu.SemaphoreType.DMA,
            pltpu.SemaphoreType.DMA,
            pltpu.SemaphoreType.DMA,
        ],
    )
    return deg, agg


def _dinv_of(degp_ref):
    deg = degp_ref[0, :, 0] + degp_ref[1, :, 0] + 1.0
    return lax.rsqrt(deg)


def _prep_body(degp_ref, x_ref, w1_ref, hs_ref):
    dinv = _dinv_of(degp_ref)
    p = jnp.dot(x_ref[...], w1_ref[...], preferred_element_type=jnp.float32)
    hs_ref[...] = p * dinv[:, None]


_prep = pl.pallas_call(
    _prep_body,
    out_shape=jax.ShapeDtypeStruct((N, H), jnp.float32),
)


def _mid_body(degp_ref, part_ref, hs_ref, b1_ref, w2_ref, hs2_ref):
    dinv = _dinv_of(degp_ref)
    agg = part_ref[0] + part_ref[1] + hs_ref[...]
    h1 = jnp.maximum(agg * dinv[:, None] + b1_ref[...], 0.0)
    hs2_ref[...] = (
        jnp.dot(h1, w2_ref[...], preferred_element_type=jnp.float32) * dinv[:, None]
    )


_mid = pl.pallas_call(
    _mid_body,
    out_shape=jax.ShapeDtypeStruct((N, H), jnp.float32),
)


def _final_body(degp_ref, part_ref, hs2_ref, b2_ref, batch_ref, sigma_ref,
                ws1_ref, bs1_ref, ws2_ref, bs2_ref, wf_ref, bf_ref,
                wp_ref, bp_ref, out_ref, acc_ref):
    i = pl.program_id(0)

    @pl.when(i == 0)
    def _():
        acc_ref[...] = jnp.zeros_like(acc_ref)

    dinv = _dinv_of(degp_ref)
    agg = part_ref[0] + part_ref[1] + hs2_ref[...]
    h2 = jnp.maximum(agg * dinv[:, None] + b2_ref[...], 0.0)
    h2e = jnp.concatenate([h2, jnp.ones((RB, 1), jnp.float32)], axis=1)
    bb = batch_ref[0, 0, :]
    onehot = (bb[:, None] == lax.broadcasted_iota(jnp.int32, (1, B), 1)
              ).astype(jnp.float32)
    acc_ref[...] += lax.dot_general(
        onehot, h2e, dimension_numbers=(((0,), (0,)), ((), ())),
        preferred_element_type=jnp.float32)

    @pl.when(i == NBLK - 1)
    def _():
        cnt = jnp.maximum(acc_ref[:, H:H + 1], 1.0)
        mol = acc_ref[:, :H] / cnt
        sf = jnp.maximum(
            jnp.dot(sigma_ref[...], ws1_ref[...],
                    preferred_element_type=jnp.float32) + bs1_ref[...], 0.0)
        sf = jnp.maximum(
            jnp.dot(sf, ws2_ref[...],
                    preferred_element_type=jnp.float32) + bs2_ref[...], 0.0)
        comb = jnp.maximum(
            jnp.dot(sf, wf_ref[0:H, :], preferred_element_type=jnp.float32)
            + jnp.dot(mol, wf_ref[H:2 * H, :], preferred_element_type=jnp.float32)
            + bf_ref[...], 0.0)
        out_ref[...] = (
            jnp.dot(comb, wp_ref[...], preferred_element_type=jnp.float32)
            + bp_ref[...])


_final = pl.pallas_call(
    _final_body,
    grid=(NBLK,),
    in_specs=[
        pl.BlockSpec((NC, RB, DEGW), lambda i: (0, i, 0)),
        pl.BlockSpec((NC, RB, H), lambda i: (0, i, 0)),
        pl.BlockSpec((RB, H), lambda i: (i, 0)),
        pl.BlockSpec((1, H), lambda i: (0, 0)),
        pl.BlockSpec((1, 1, RB), lambda i: (i, 0, 0)),
        pl.BlockSpec((B, SD), lambda i: (0, 0)),
        pl.BlockSpec((SD, 2 * H), lambda i: (0, 0)),
        pl.BlockSpec((1, 2 * H), lambda i: (0, 0)),
        pl.BlockSpec((2 * H, H), lambda i: (0, 0)),
        pl.BlockSpec((1, H), lambda i: (0, 0)),
        pl.BlockSpec((2 * H, H), lambda i: (0, 0)),
        pl.BlockSpec((1, H), lambda i: (0, 0)),
        pl.BlockSpec((H, 1), lambda i: (0, 0)),
        pl.BlockSpec((1, 1), lambda i: (0, 0)),
    ],
    out_specs=pl.BlockSpec((B, 1), lambda i: (0, 0)),
    out_shape=jax.ShapeDtypeStruct((B, 1), jnp.float32),
    scratch_shapes=[pltpu.VMEM((B, H + 1), jnp.float32)],
)


def kernel(sigma_data, x, edge_index, batch, W_g1, b_g1, W_g2, b_g2,
           W_s1, b_s1, W_s2, b_s2, W_f, b_f, W_p, b_p):
    src2 = edge_index[0].reshape(NW, EPW)
    dst2 = edge_index[1].reshape(NW, EPW)
    pad = ((0, 0), (0, EPAD - EPW))
    src3 = jnp.pad(src2, pad).reshape(NW, NCHUNK, CHUNK)
    dst3 = jnp.pad(dst2, pad, constant_values=SINK).reshape(NW, NCHUNK, CHUNK)
    zeros_h = jnp.zeros((NPAD, H), jnp.float32)
    zeros_d = jnp.zeros((NPAD, DEGW), jnp.float32)
    ones_d = jnp.ones((CHUNK, DEGW), jnp.float32)
    batch3 = batch.reshape(NBLK, 1, RB)

    _deg_kernel, _agg_kernel = _sc_kernels()
    degp = _deg_kernel(dst3, zeros_d, ones_d)
    hs1 = _prep(degp, x, W_g1)
    part1 = _agg_kernel(hs1, src3, dst3, zeros_h)
    hs2 = _mid(degp, part1, hs1, b_g1.reshape(1, H), W_g2)
    part2 = _agg_kernel(hs2, src3, dst3, zeros_h)
    pred = _final(degp, part2, hs2, b_g2.reshape(1, H), batch3, sigma_data,
                  W_s1, b_s1.reshape(1, 2 * H), W_s2, b_s2.reshape(1, H),
                  W_f, b_f.reshape(1, H), W_p, b_p.reshape(1, 1))
    return pred.reshape(B)
